# baseline (speedup 1.0000x reference)
import functools

import jax
import jax.numpy as jnp
from jax import lax
from jax.experimental import pallas as pl
from jax.experimental.pallas import tpu as pltpu
from jax.experimental.pallas import tpu_sc as plsc

_ROT0 = (13, 15, 26, 6)
_ROT1 = (17, 29, 16, 24)
_KS0 = 0
_KS1 = 42
_KS2 = _KS0 ^ _KS1 ^ 0x1BD11BDA
_TINY = 1.1754943508222875e-38

_B = 128
_V = 100000
_VTC = 70656
_VSC = _V - _VTC
_VSCP = 29440


def _rotl(x, d):
    return (x << jnp.uint32(d)) | (x >> jnp.uint32(32 - d))


def _threefry_bits(p):
    ks = (jnp.uint32(_KS0), jnp.uint32(_KS1), jnp.uint32(_KS2))
    x0 = jnp.zeros_like(p) + ks[0]
    x1 = p + ks[1]
    for i in range(5):
        rots = _ROT0 if i % 2 == 0 else _ROT1
        for r in rots:
            x0 = x0 + x1
            x1 = _rotl(x1, r)
            x1 = x0 ^ x1
        x0 = x0 + ks[(i + 1) % 3]
        x1 = x1 + ks[(i + 2) % 3] + jnp.uint32(i + 1)
    return x0 ^ x1


def _uniform_from_bits(bits):
    float_bits = (bits >> jnp.uint32(9)) | jnp.uint32(0x3F800000)
    floats = lax.bitcast_convert_type(float_bits, jnp.float32) - jnp.float32(1.0)
    return jnp.maximum(jnp.float32(_TINY), floats + jnp.float32(_TINY))


def _gumbel_from_u(u):
    return -jnp.log(-jnp.log(u))


@functools.cache
def _make_sc_uniform():
    mesh = plsc.VectorSubcoreMesh(core_axis_name="c", subcore_axis_name="s")

    @functools.partial(
        pl.kernel,
        out_type=jax.ShapeDtypeStruct((_B, _VSCP), jnp.float32),
        mesh=mesh,
        scratch_types=[pltpu.VMEM((_VSCP,), jnp.float32)],
    )
    def _sc_uniform(out_hbm, row_buf):
        wid = lax.axis_index("s") * 2 + lax.axis_index("c")
        lane = lax.iota(jnp.int32, 16)
        rows_per_w = _B // 32

        def do_row(r, _):
            row = wid * rows_per_w + r
            base = row * _V + _VTC

            def chunk(i, _):
                p = (base + i * 16 + lane).astype(jnp.uint32)
                row_buf[pl.ds(i * 16, 16)] = _uniform_from_bits(
                    _threefry_bits(p))
                return 0

            lax.fori_loop(0, _VSCP // 16, chunk, 0, unroll=4)
            pltpu.sync_copy(row_buf, out_hbm.at[row])
            return 0

        lax.fori_loop(0, rows_per_w, do_row, 0)

    return _sc_uniform


def _tc_main_kernel(logits_ref, val_ref, idx_ref, *, block_rows, chunk):
    row0 = pl.program_id(0) * block_rows
    nchunks = _VTC // chunk
    rows = lax.broadcasted_iota(jnp.int32, (block_rows, chunk), 0) + row0
    base_cols = lax.broadcasted_iota(jnp.int32, (block_rows, chunk), 1)
    row_off = rows * _V

    def body(c, carry):
        acc_val, acc_col = carry
        start = c * chunk
        cols = base_cols + start
        p = (row_off + cols).astype(jnp.uint32)
        g = _gumbel_from_u(_uniform_from_bits(_threefry_bits(p)))
        vals = g + logits_ref[:, pl.ds(start, chunk)]
        better = vals > acc_val
        acc_val = jnp.where(better, vals, acc_val)
        acc_col = jnp.where(better, cols, acc_col)
        return acc_val, acc_col

    init = (jnp.full((block_rows, chunk), -jnp.inf, jnp.float32),
            jnp.zeros((block_rows, chunk), jnp.int32))
    acc_val, acc_col = lax.fori_loop(0, nchunks, body, init, unroll=4)

    m = jnp.max(acc_val, axis=1, keepdims=True)
    idx = jnp.min(jnp.where(acc_val == m, acc_col, jnp.int32(2**31 - 1)),
                  axis=1, keepdims=True)
    val_ref[...] = m
    idx_ref[...] = idx


def _tc_merge_kernel(logits_sc_ref, u_ref, pval_ref, pidx_ref, out_ref, *,
                     block_rows, chunk):
    base_cols = lax.broadcasted_iota(jnp.int32, (block_rows, chunk), 1)

    def body(c, carry):
        acc_val, acc_col = carry
        start = jnp.minimum(c * chunk, _VSCP - chunk)
        cols = base_cols + start
        vals = (_gumbel_from_u(u_ref[:, pl.ds(start, chunk)])
                + logits_sc_ref[:, pl.ds(start, chunk)])
        vals = jnp.where(cols < _VSC, vals, -jnp.inf)
        better = vals > acc_val
        acc_val = jnp.where(better, vals, acc_val)
        acc_col = jnp.where(better, cols, acc_col)
        return acc_val, acc_col

    init = (jnp.full((block_rows, chunk), -jnp.inf, jnp.float32),
            jnp.zeros((block_rows, chunk), jnp.int32))
    acc_val, acc_col = lax.fori_loop(0, pl.cdiv(_VSCP, chunk), body, init,
                                     unroll=4)

    m2 = jnp.max(acc_val, axis=1, keepdims=True)
    idx2 = jnp.min(jnp.where(acc_val == m2, acc_col, jnp.int32(2**31 - 1)),
                   axis=1, keepdims=True) + _VTC
    win2 = m2 > pval_ref[...]
    out_ref[...] = jnp.where(win2, idx2, pidx_ref[...])


@jax.jit
def kernel(logits):
    b, vocab = logits.shape
    block_rows = 8
    chunk = 1024
    grid = (b // block_rows,)

    u_sc = _make_sc_uniform()()

    pval, pidx = pl.pallas_call(
        functools.partial(_tc_main_kernel, block_rows=block_rows, chunk=chunk),
        grid=grid,
        in_specs=[pl.BlockSpec((block_rows, _VTC), lambda i: (i, 0))],
        out_specs=[pl.BlockSpec((block_rows, 1), lambda i: (i, 0)),
                   pl.BlockSpec((block_rows, 1), lambda i: (i, 0))],
        out_shape=[jax.ShapeDtypeStruct((b, 1), jnp.float32),
                   jax.ShapeDtypeStruct((b, 1), jnp.int32)],
        compiler_params=pltpu.CompilerParams(
            dimension_semantics=("arbitrary",),
        ),
    )(logits[:, :_VTC])

    out = pl.pallas_call(
        functools.partial(_tc_merge_kernel, block_rows=block_rows,
                          chunk=chunk),
        grid=grid,
        in_specs=[pl.BlockSpec((block_rows, _VSCP), lambda i: (i, 0)),
                  pl.BlockSpec((block_rows, _VSCP), lambda i: (i, 0)),
                  pl.BlockSpec((block_rows, 1), lambda i: (i, 0)),
                  pl.BlockSpec((block_rows, 1), lambda i: (i, 0))],
        out_specs=pl.BlockSpec((block_rows, 1), lambda i: (i, 0)),
        out_shape=jax.ShapeDtypeStruct((b, 1), jnp.int32),
        compiler_params=pltpu.CompilerParams(
            dimension_semantics=("arbitrary",),
        ),
    )(jnp.pad(logits[:, _VTC:], ((0, 0), (0, _VSCP - _VSC))), u_sc, pval,
      pidx)

    return out[:, 0].astype(jnp.int64)


# speedup vs baseline: 1.1543x; 1.1543x over previous
import functools

import jax
import jax.numpy as jnp
from jax import lax
from jax.experimental import pallas as pl
from jax.experimental.pallas import tpu as pltpu
from jax.experimental.pallas import tpu_sc as plsc

_ROT0 = (13, 15, 26, 6)
_ROT1 = (17, 29, 16, 24)
_KS0 = 0
_KS1 = 42
_KS2 = _KS0 ^ _KS1 ^ 0x1BD11BDA
_TINY = 1.1754943508222875e-38

_B = 128
_V = 100000
_SC_W = 29440
_TC_START = 28672
_CHUNK = 1024


def _rotl(x, d):
    return (x << jnp.uint32(d)) | (x >> jnp.uint32(32 - d))


def _threefry_bits(p):
    ks = (jnp.uint32(_KS0), jnp.uint32(_KS1), jnp.uint32(_KS2))
    x0 = jnp.zeros_like(p) + ks[0]
    x1 = p + ks[1]
    for i in range(5):
        rots = _ROT0 if i % 2 == 0 else _ROT1
        for r in rots:
            x0 = x0 + x1
            x1 = _rotl(x1, r)
            x1 = x0 ^ x1
        x0 = x0 + ks[(i + 1) % 3]
        x1 = x1 + ks[(i + 2) % 3] + jnp.uint32(i + 1)
    return x0 ^ x1


def _uniform_from_bits(bits):
    float_bits = (bits >> jnp.uint32(9)) | jnp.uint32(0x3F800000)
    floats = lax.bitcast_convert_type(float_bits, jnp.float32) - jnp.float32(1.0)
    return jnp.maximum(jnp.float32(_TINY), floats + jnp.float32(_TINY))


def _gumbel_from_u(u):
    return -jnp.log(-jnp.log(u))


@functools.cache
def _make_sc_uniform():
    mesh = plsc.VectorSubcoreMesh(core_axis_name="c", subcore_axis_name="s")

    @functools.partial(
        pl.kernel,
        out_type=jax.ShapeDtypeStruct((_B, _SC_W), jnp.float32),
        mesh=mesh,
        scratch_types=[pltpu.VMEM((_SC_W,), jnp.float32)],
    )
    def _sc_uniform(out_hbm, row_buf):
        wid = lax.axis_index("s") * 2 + lax.axis_index("c")
        lane = lax.iota(jnp.int32, 16)
        rows_per_w = _B // 32

        def do_row(r, _):
            row = wid * rows_per_w + r
            base = row * _V

            def chunk(i, _):
                p = (base + i * 16 + lane).astype(jnp.uint32)
                row_buf[pl.ds(i * 16, 16)] = _uniform_from_bits(
                    _threefry_bits(p))
                return 0

            lax.fori_loop(0, _SC_W // 16, chunk, 0, unroll=4)
            pltpu.sync_copy(row_buf, out_hbm.at[row])
            return 0

        lax.fori_loop(0, rows_per_w, do_row, 0)

    return _sc_uniform


def _tc_main_kernel(logits_ref, val_ref, idx_ref, *, block_rows, vpad):
    row0 = pl.program_id(0) * block_rows
    span = vpad - _TC_START
    nchunks = pl.cdiv(span, _CHUNK)
    rows = lax.broadcasted_iota(jnp.int32, (block_rows, _CHUNK), 0) + row0
    base_cols = lax.broadcasted_iota(jnp.int32, (block_rows, _CHUNK), 1)
    row_off = rows * _V

    def body(c, carry):
        acc_val, acc_col = carry
        start = jnp.minimum(_TC_START + c * _CHUNK, vpad - _CHUNK)
        cols = base_cols + start
        p = (row_off + cols).astype(jnp.uint32)
        g = _gumbel_from_u(_uniform_from_bits(_threefry_bits(p)))
        vals = g + logits_ref[:, pl.ds(start, _CHUNK)]
        vals = jnp.where(cols < _V, vals, -jnp.inf)
        better = vals > acc_val
        acc_val = jnp.where(better, vals, acc_val)
        acc_col = jnp.where(better, cols, acc_col)
        return acc_val, acc_col

    init = (jnp.full((block_rows, _CHUNK), -jnp.inf, jnp.float32),
            jnp.zeros((block_rows, _CHUNK), jnp.int32))
    acc_val, acc_col = lax.fori_loop(0, nchunks, body, init, unroll=4)

    m = jnp.max(acc_val, axis=1, keepdims=True)
    idx = jnp.min(jnp.where(acc_val == m, acc_col, jnp.int32(2**31 - 1)),
                  axis=1, keepdims=True)
    val_ref[...] = m
    idx_ref[...] = idx


def _tc_merge_kernel(logits_ref, u_ref, pval_ref, pidx_ref, out_ref, *,
                     block_rows):
    base_cols = lax.broadcasted_iota(jnp.int32, (block_rows, _CHUNK), 1)

    def body(c, carry):
        acc_val, acc_col = carry
        start = jnp.minimum(c * _CHUNK, _SC_W - _CHUNK)
        cols = base_cols + start
        vals = (_gumbel_from_u(u_ref[:, pl.ds(start, _CHUNK)])
                + logits_ref[:, pl.ds(start, _CHUNK)])
        better = vals > acc_val
        acc_val = jnp.where(better, vals, acc_val)
        acc_col = jnp.where(better, cols, acc_col)
        return acc_val, acc_col

    init = (jnp.full((block_rows, _CHUNK), -jnp.inf, jnp.float32),
            jnp.zeros((block_rows, _CHUNK), jnp.int32))
    acc_val, acc_col = lax.fori_loop(0, pl.cdiv(_SC_W, _CHUNK), body, init,
                                     unroll=4)

    m2 = jnp.max(acc_val, axis=1, keepdims=True)
    idx2 = jnp.min(jnp.where(acc_val == m2, acc_col, jnp.int32(2**31 - 1)),
                   axis=1, keepdims=True)
    win2 = m2 >= pval_ref[...]
    out_ref[...] = jnp.where(win2, idx2, pidx_ref[...])


@jax.jit
def kernel(logits):
    b, vocab = logits.shape
    block_rows = 8
    grid = (b // block_rows,)
    vpad = pl.cdiv(vocab, 128) * 128

    u_sc = _make_sc_uniform()()

    pval, pidx = pl.pallas_call(
        functools.partial(_tc_main_kernel, block_rows=block_rows, vpad=vpad),
        grid=grid,
        in_specs=[pl.BlockSpec((block_rows, vocab), lambda i: (i, 0))],
        out_specs=[pl.BlockSpec((block_rows, 1), lambda i: (i, 0)),
                   pl.BlockSpec((block_rows, 1), lambda i: (i, 0))],
        out_shape=[jax.ShapeDtypeStruct((b, 1), jnp.float32),
                   jax.ShapeDtypeStruct((b, 1), jnp.int32)],
        compiler_params=pltpu.CompilerParams(
            dimension_semantics=("arbitrary",),
        ),
    )(logits)

    out = pl.pallas_call(
        functools.partial(_tc_merge_kernel, block_rows=block_rows),
        grid=grid,
        in_specs=[pl.BlockSpec((block_rows, _SC_W), lambda i: (i, 0)),
                  pl.BlockSpec((block_rows, _SC_W), lambda i: (i, 0)),
                  pl.BlockSpec((block_rows, 1), lambda i: (i, 0)),
                  pl.BlockSpec((block_rows, 1), lambda i: (i, 0))],
        out_specs=pl.BlockSpec((block_rows, 1), lambda i: (i, 0)),
        out_shape=jax.ShapeDtypeStruct((b, 1), jnp.int32),
        compiler_params=pltpu.CompilerParams(
            dimension_semantics=("arbitrary",),
        ),
    )(logits, u_sc, pval, pidx)

    return out[:, 0].astype(jnp.int64)


# speedup vs baseline: 1.2161x; 1.0535x over previous
import functools

import jax
import jax.numpy as jnp
from jax import lax
from jax.experimental import pallas as pl
from jax.experimental.pallas import tpu as pltpu
from jax.experimental.pallas import tpu_sc as plsc

_ROT0 = (13, 15, 26, 6)
_ROT1 = (17, 29, 16, 24)
_KS0 = 0
_KS1 = 42
_KS2 = _KS0 ^ _KS1 ^ 0x1BD11BDA
_TINY = 1.1754943508222875e-38

_B = 128
_V = 100000
_SC_W = 33920
_TC_START = 33792
_CHUNK = 1024


def _rotl(x, d):
    return (x << jnp.uint32(d)) | (x >> jnp.uint32(32 - d))


def _threefry_bits(p):
    ks = (jnp.uint32(_KS0), jnp.uint32(_KS1), jnp.uint32(_KS2))
    x1 = p + ks[1]
    x0 = x1
    x1 = x0 ^ _rotl(x1, _ROT0[0])
    for r in _ROT0[1:]:
        x0 = x0 + x1
        x1 = x0 ^ _rotl(x1, r)
    x0 = x0 + ks[1]
    x1 = x1 + ks[2] + jnp.uint32(1)
    for i in range(1, 5):
        rots = _ROT0 if i % 2 == 0 else _ROT1
        for r in rots:
            x0 = x0 + x1
            x1 = x0 ^ _rotl(x1, r)
        x0 = x0 + ks[(i + 1) % 3]
        x1 = x1 + ks[(i + 2) % 3] + jnp.uint32(i + 1)
    return x0 ^ x1


def _uniform_from_bits(bits):
    float_bits = (bits >> jnp.uint32(9)) | jnp.uint32(0x3F800000)
    floats = lax.bitcast_convert_type(float_bits, jnp.float32) - jnp.float32(1.0)
    return jnp.maximum(jnp.float32(_TINY), floats + jnp.float32(_TINY))


def _gumbel_from_u(u):
    return -jnp.log(-jnp.log(u))


@functools.cache
def _make_sc_uniform():
    mesh = plsc.VectorSubcoreMesh(core_axis_name="c", subcore_axis_name="s")

    @functools.partial(
        pl.kernel,
        out_type=jax.ShapeDtypeStruct((_B, _SC_W), jnp.float32),
        mesh=mesh,
        scratch_types=[pltpu.VMEM((_SC_W,), jnp.float32)],
    )
    def _sc_uniform(out_hbm, row_buf):
        wid = lax.axis_index("s") * 2 + lax.axis_index("c")
        lane = lax.iota(jnp.int32, 16)
        rows_per_w = _B // 32

        def do_row(r, _):
            row = wid * rows_per_w + r
            base = row * _V

            def chunk(i, _):
                p = (base + i * 16 + lane).astype(jnp.uint32)
                row_buf[pl.ds(i * 16, 16)] = _uniform_from_bits(
                    _threefry_bits(p))
                return 0

            lax.fori_loop(0, _SC_W // 16, chunk, 0, unroll=8)
            pltpu.sync_copy(row_buf, out_hbm.at[row])
            return 0

        lax.fori_loop(0, rows_per_w, do_row, 0)

    return _sc_uniform


def _tc_main_kernel(logits_ref, val_ref, idx_ref, *, block_rows, vpad):
    row0 = pl.program_id(0) * block_rows
    span = vpad - _TC_START
    nchunks = pl.cdiv(span, _CHUNK)
    rows = lax.broadcasted_iota(jnp.int32, (block_rows, _CHUNK), 0) + row0
    base_cols = lax.broadcasted_iota(jnp.int32, (block_rows, _CHUNK), 1)
    row_off = rows * _V

    def step(start, carry, masked):
        acc_val, acc_col = carry
        cols = base_cols + start
        p = (row_off + cols).astype(jnp.uint32)
        g = _gumbel_from_u(_uniform_from_bits(_threefry_bits(p)))
        vals = g + logits_ref[:, pl.ds(start, _CHUNK)]
        if masked:
            vals = jnp.where(cols < _V, vals, -jnp.inf)
        better = vals > acc_val
        acc_val = jnp.where(better, vals, acc_val)
        acc_col = jnp.where(better, cols, acc_col)
        return acc_val, acc_col

    init = (jnp.full((block_rows, _CHUNK), -jnp.inf, jnp.float32),
            jnp.zeros((block_rows, _CHUNK), jnp.int32))
    acc_val, acc_col = lax.fori_loop(
        0, nchunks - 1,
        lambda c, carry: step(_TC_START + c * _CHUNK, carry, masked=False),
        init, unroll=4)
    acc_val, acc_col = lax.fori_loop(
        nchunks - 1, nchunks,
        lambda c, carry: step(
            jnp.minimum(_TC_START + c * _CHUNK, vpad - _CHUNK), carry,
            masked=True),
        (acc_val, acc_col))

    m = jnp.max(acc_val, axis=1, keepdims=True)
    idx = jnp.min(jnp.where(acc_val == m, acc_col, jnp.int32(2**31 - 1)),
                  axis=1, keepdims=True)
    val_ref[...] = m
    idx_ref[...] = idx


def _tc_merge_kernel(logits_ref, u_ref, pval_ref, pidx_ref, out_ref, *,
                     block_rows):
    base_cols = lax.broadcasted_iota(jnp.int32, (block_rows, _CHUNK), 1)

    def body(c, carry):
        acc_val, acc_col = carry
        start = jnp.minimum(c * _CHUNK, _SC_W - _CHUNK)
        cols = base_cols + start
        vals = (_gumbel_from_u(u_ref[:, pl.ds(start, _CHUNK)])
                + logits_ref[:, pl.ds(start, _CHUNK)])
        better = vals > acc_val
        acc_val = jnp.where(better, vals, acc_val)
        acc_col = jnp.where(better, cols, acc_col)
        return acc_val, acc_col

    init = (jnp.full((block_rows, _CHUNK), -jnp.inf, jnp.float32),
            jnp.zeros((block_rows, _CHUNK), jnp.int32))
    acc_val, acc_col = lax.fori_loop(0, pl.cdiv(_SC_W, _CHUNK), body, init,
                                     unroll=4)

    m2 = jnp.max(acc_val, axis=1, keepdims=True)
    idx2 = jnp.min(jnp.where(acc_val == m2, acc_col, jnp.int32(2**31 - 1)),
                   axis=1, keepdims=True)
    win2 = m2 >= pval_ref[...]
    out_ref[...] = jnp.where(win2, idx2, pidx_ref[...])


@jax.jit
def kernel(logits):
    b, vocab = logits.shape
    block_rows = 8
    grid = (b // block_rows,)
    vpad = pl.cdiv(vocab, 128) * 128

    u_sc = _make_sc_uniform()()

    pval, pidx = pl.pallas_call(
        functools.partial(_tc_main_kernel, block_rows=block_rows, vpad=vpad),
        grid=grid,
        in_specs=[pl.BlockSpec((block_rows, vocab), lambda i: (i, 0))],
        out_specs=[pl.BlockSpec((block_rows, 1), lambda i: (i, 0)),
                   pl.BlockSpec((block_rows, 1), lambda i: (i, 0))],
        out_shape=[jax.ShapeDtypeStruct((b, 1), jnp.float32),
                   jax.ShapeDtypeStruct((b, 1), jnp.int32)],
        compiler_params=pltpu.CompilerParams(
            dimension_semantics=("arbitrary",),
        ),
    )(logits)

    out = pl.pallas_call(
        functools.partial(_tc_merge_kernel, block_rows=block_rows),
        grid=grid,
        in_specs=[pl.BlockSpec((block_rows, _SC_W), lambda i: (i, 0)),
                  pl.BlockSpec((block_rows, _SC_W), lambda i: (i, 0)),
                  pl.BlockSpec((block_rows, 1), lambda i: (i, 0)),
                  pl.BlockSpec((block_rows, 1), lambda i: (i, 0))],
        out_specs=pl.BlockSpec((block_rows, 1), lambda i: (i, 0)),
        out_shape=jax.ShapeDtypeStruct((b, 1), jnp.int32),
        compiler_params=pltpu.CompilerParams(
            dimension_semantics=("arbitrary",),
        ),
    )(logits, u_sc, pval, pidx)

    return out[:, 0].astype(jnp.int64)


# speedup vs baseline: 1.2198x; 1.0031x over previous
import functools

import jax
import jax.numpy as jnp
from jax import lax
from jax.experimental import pallas as pl
from jax.experimental.pallas import tpu as pltpu
from jax.experimental.pallas import tpu_sc as plsc

_ROT0 = (13, 15, 26, 6)
_ROT1 = (17, 29, 16, 24)
_KS0 = 0
_KS1 = 42
_KS2 = _KS0 ^ _KS1 ^ 0x1BD11BDA
_TINY = 1.1754943508222875e-38

_B = 128
_V = 100000
_SC_W = 33920
_TC_START = 33792
_CHUNK = 1024


def _rotl(x, d):
    return (x << jnp.uint32(d)) | (x >> jnp.uint32(32 - d))


def _threefry_bits(p):
    ks = (jnp.uint32(_KS0), jnp.uint32(_KS1), jnp.uint32(_KS2))
    x1 = p + ks[1]
    x0 = x1
    x1 = x0 ^ _rotl(x1, _ROT0[0])
    for r in _ROT0[1:]:
        x0 = x0 + x1
        x1 = x0 ^ _rotl(x1, r)
    x0 = x0 + ks[1]
    x1 = x1 + ks[2] + jnp.uint32(1)
    for i in range(1, 5):
        rots = _ROT0 if i % 2 == 0 else _ROT1
        for r in rots:
            x0 = x0 + x1
            x1 = x0 ^ _rotl(x1, r)
        x0 = x0 + ks[(i + 1) % 3]
        x1 = x1 + ks[(i + 2) % 3] + jnp.uint32(i + 1)
    return x0 ^ x1


def _uniform_from_bits(bits):
    float_bits = (bits >> jnp.uint32(9)) | jnp.uint32(0x3F800000)
    floats = lax.bitcast_convert_type(float_bits, jnp.float32) - jnp.float32(1.0)
    return jnp.maximum(jnp.float32(_TINY), floats + jnp.float32(_TINY))


def _gumbel_from_u(u):
    return -jnp.log(-jnp.log(u))


@functools.cache
def _make_sc_uniform():
    mesh = plsc.VectorSubcoreMesh(core_axis_name="c", subcore_axis_name="s")

    @functools.partial(
        pl.kernel,
        out_type=jax.ShapeDtypeStruct((_B, _SC_W), jnp.float32),
        mesh=mesh,
        scratch_types=[pltpu.VMEM((_SC_W,), jnp.float32)],
    )
    def _sc_uniform(out_hbm, row_buf):
        wid = lax.axis_index("s") * 2 + lax.axis_index("c")
        lane = lax.iota(jnp.int32, 16)
        rows_per_w = _B // 32

        def do_row(r, _):
            row = wid * rows_per_w + r
            base = row * _V

            def chunk(i, _):
                p = (base + i * 16 + lane).astype(jnp.uint32)
                row_buf[pl.ds(i * 16, 16)] = _uniform_from_bits(
                    _threefry_bits(p))
                return 0

            lax.fori_loop(0, _SC_W // 16, chunk, 0, unroll=8)
            pltpu.sync_copy(row_buf, out_hbm.at[row])
            return 0

        lax.fori_loop(0, rows_per_w, do_row, 0)

    return _sc_uniform


def _tc_main_kernel(logits_ref, val_ref, idx_ref, *, block_rows, vpad):
    row0 = pl.program_id(0) * block_rows
    span = vpad - _TC_START
    nchunks = pl.cdiv(span, _CHUNK)
    rows = lax.broadcasted_iota(jnp.int32, (block_rows, _CHUNK), 0) + row0
    base_cols = lax.broadcasted_iota(jnp.int32, (block_rows, _CHUNK), 1)
    row_off = rows * _V

    def step(start, carry, masked):
        acc_val, acc_col = carry
        cols = base_cols + start
        p = (row_off + cols).astype(jnp.uint32)
        g = _gumbel_from_u(_uniform_from_bits(_threefry_bits(p)))
        vals = g + logits_ref[:, pl.ds(start, _CHUNK)]
        if masked:
            vals = jnp.where(cols < _V, vals, -jnp.inf)
        better = vals > acc_val
        acc_val = jnp.where(better, vals, acc_val)
        acc_col = jnp.where(better, cols, acc_col)
        return acc_val, acc_col

    init = (jnp.full((block_rows, _CHUNK), -jnp.inf, jnp.float32),
            jnp.zeros((block_rows, _CHUNK), jnp.int32))
    acc_val, acc_col = lax.fori_loop(
        0, nchunks - 1,
        lambda c, carry: step(_TC_START + c * _CHUNK, carry, masked=False),
        init, unroll=8)
    acc_val, acc_col = lax.fori_loop(
        nchunks - 1, nchunks,
        lambda c, carry: step(
            jnp.minimum(_TC_START + c * _CHUNK, vpad - _CHUNK), carry,
            masked=True),
        (acc_val, acc_col))

    m = jnp.max(acc_val, axis=1, keepdims=True)
    idx = jnp.min(jnp.where(acc_val == m, acc_col, jnp.int32(2**31 - 1)),
                  axis=1, keepdims=True)
    val_ref[...] = m
    idx_ref[...] = idx


def _tc_merge_kernel(logits_ref, u_ref, pval_ref, pidx_ref, out_ref, *,
                     block_rows):
    base_cols = lax.broadcasted_iota(jnp.int32, (block_rows, _CHUNK), 1)

    def body(c, carry):
        acc_val, acc_col = carry
        start = jnp.minimum(c * _CHUNK, _SC_W - _CHUNK)
        cols = base_cols + start
        vals = (_gumbel_from_u(u_ref[:, pl.ds(start, _CHUNK)])
                + logits_ref[:, pl.ds(start, _CHUNK)])
        better = vals > acc_val
        acc_val = jnp.where(better, vals, acc_val)
        acc_col = jnp.where(better, cols, acc_col)
        return acc_val, acc_col

    init = (jnp.full((block_rows, _CHUNK), -jnp.inf, jnp.float32),
            jnp.zeros((block_rows, _CHUNK), jnp.int32))
    acc_val, acc_col = lax.fori_loop(0, pl.cdiv(_SC_W, _CHUNK), body, init,
                                     unroll=8)

    m2 = jnp.max(acc_val, axis=1, keepdims=True)
    idx2 = jnp.min(jnp.where(acc_val == m2, acc_col, jnp.int32(2**31 - 1)),
                   axis=1, keepdims=True)
    win2 = m2 >= pval_ref[...]
    out_ref[...] = jnp.where(win2, idx2, pidx_ref[...])


@jax.jit
def kernel(logits):
    b, vocab = logits.shape
    block_rows = 8
    grid = (b // block_rows,)
    vpad = pl.cdiv(vocab, 128) * 128

    u_sc = _make_sc_uniform()()

    pval, pidx = pl.pallas_call(
        functools.partial(_tc_main_kernel, block_rows=block_rows, vpad=vpad),
        grid=grid,
        in_specs=[pl.BlockSpec((block_rows, vocab), lambda i: (i, 0))],
        out_specs=[pl.BlockSpec((block_rows, 1), lambda i: (i, 0)),
                   pl.BlockSpec((block_rows, 1), lambda i: (i, 0))],
        out_shape=[jax.ShapeDtypeStruct((b, 1), jnp.float32),
                   jax.ShapeDtypeStruct((b, 1), jnp.int32)],
        compiler_params=pltpu.CompilerParams(
            dimension_semantics=("arbitrary",),
        ),
    )(logits)

    out = pl.pallas_call(
        functools.partial(_tc_merge_kernel, block_rows=block_rows),
        grid=grid,
        in_specs=[pl.BlockSpec((block_rows, _SC_W), lambda i: (i, 0)),
                  pl.BlockSpec((block_rows, _SC_W), lambda i: (i, 0)),
                  pl.BlockSpec((block_rows, 1), lambda i: (i, 0)),
                  pl.BlockSpec((block_rows, 1), lambda i: (i, 0))],
        out_specs=pl.BlockSpec((block_rows, 1), lambda i: (i, 0)),
        out_shape=jax.ShapeDtypeStruct((b, 1), jnp.int32),
        compiler_params=pltpu.CompilerParams(
            dimension_semantics=("arbitrary",),
        ),
    )(logits, u_sc, pval, pidx)

    return out[:, 0].astype(jnp.int64)


# speedup vs baseline: 1.2374x; 1.0145x over previous
import functools

import jax
import jax.numpy as jnp
from jax import lax
from jax.experimental import pallas as pl
from jax.experimental.pallas import tpu as pltpu
from jax.experimental.pallas import tpu_sc as plsc

_ROT0 = (13, 15, 26, 6)
_ROT1 = (17, 29, 16, 24)
_KS0 = 0
_KS1 = 42
_KS2 = _KS0 ^ _KS1 ^ 0x1BD11BDA
_TINY = 1.1754943508222875e-38

_B = 128
_V = 100000
_SC_W = 33920
_TC_START = 33792
_CHUNK = 1024


def _rotl(x, d):
    return (x << jnp.uint32(d)) | (x >> jnp.uint32(32 - d))


def _threefry_bits(p):
    ks = (jnp.uint32(_KS0), jnp.uint32(_KS1), jnp.uint32(_KS2))
    x1 = p + ks[1]
    x0 = x1
    x1 = x0 ^ _rotl(x1, _ROT0[0])
    for r in _ROT0[1:]:
        x0 = x0 + x1
        x1 = x0 ^ _rotl(x1, r)
    x0 = x0 + ks[1]
    x1 = x1 + ks[2] + jnp.uint32(1)
    for i in range(1, 5):
        rots = _ROT0 if i % 2 == 0 else _ROT1
        for r in rots:
            x0 = x0 + x1
            x1 = x0 ^ _rotl(x1, r)
        x0 = x0 + ks[(i + 1) % 3]
        x1 = x1 + ks[(i + 2) % 3] + jnp.uint32(i + 1)
    return x0 ^ x1


def _uniform_from_bits(bits):
    float_bits = (bits >> jnp.uint32(9)) | jnp.uint32(0x3F800000)
    floats = lax.bitcast_convert_type(float_bits, jnp.float32) - jnp.float32(1.0)
    return jnp.maximum(jnp.float32(_TINY), floats + jnp.float32(_TINY))


def _gumbel_from_u(u):
    return -jnp.log(-jnp.log(u))


@functools.cache
def _make_sc_uniform():
    mesh = plsc.VectorSubcoreMesh(core_axis_name="c", subcore_axis_name="s")

    @functools.partial(
        pl.kernel,
        out_type=jax.ShapeDtypeStruct((_B, _SC_W), jnp.float32),
        mesh=mesh,
        scratch_types=[pltpu.VMEM((_SC_W,), jnp.float32)],
    )
    def _sc_uniform(out_hbm, row_buf):
        wid = lax.axis_index("s") * 2 + lax.axis_index("c")
        lane = lax.iota(jnp.int32, 16)
        rows_per_w = _B // 32

        def do_row(r, _):
            row = wid * rows_per_w + r
            base = row * _V

            def chunk(i, _):
                p = (base + i * 16 + lane).astype(jnp.uint32)
                row_buf[pl.ds(i * 16, 16)] = _uniform_from_bits(
                    _threefry_bits(p))
                return 0

            lax.fori_loop(0, _SC_W // 16, chunk, 0, unroll=8)
            pltpu.sync_copy(row_buf, out_hbm.at[row])
            return 0

        lax.fori_loop(0, rows_per_w, do_row, 0)

    return _sc_uniform


def _tc_main_kernel(logits_ref, val_ref, idx_ref, *, block_rows, vpad):
    row0 = pl.program_id(0) * block_rows
    span = vpad - _TC_START
    nchunks = pl.cdiv(span, _CHUNK)
    rows = lax.broadcasted_iota(jnp.int32, (block_rows, _CHUNK), 0) + row0
    base_cols = lax.broadcasted_iota(jnp.int32, (block_rows, _CHUNK), 1)
    row_off = rows * _V

    def step(start, carry, masked):
        acc_val, acc_col = carry
        cols = base_cols + start
        p = (row_off + cols).astype(jnp.uint32)
        g = _gumbel_from_u(_uniform_from_bits(_threefry_bits(p)))
        vals = g + logits_ref[:, pl.ds(start, _CHUNK)]
        if masked:
            vals = jnp.where(cols < _V, vals, -jnp.inf)
        better = vals > acc_val
        acc_val = jnp.where(better, vals, acc_val)
        acc_col = jnp.where(better, cols, acc_col)
        return acc_val, acc_col

    init = (jnp.full((block_rows, _CHUNK), -jnp.inf, jnp.float32),
            jnp.zeros((block_rows, _CHUNK), jnp.int32))
    acc_val, acc_col = lax.fori_loop(
        0, nchunks - 1,
        lambda c, carry: step(_TC_START + c * _CHUNK, carry, masked=False),
        init, unroll=8)
    acc_val, acc_col = lax.fori_loop(
        nchunks - 1, nchunks,
        lambda c, carry: step(
            jnp.minimum(_TC_START + c * _CHUNK, vpad - _CHUNK), carry,
            masked=True),
        (acc_val, acc_col))

    m = jnp.max(acc_val, axis=1, keepdims=True)
    idx = jnp.min(jnp.where(acc_val == m, acc_col, jnp.int32(2**31 - 1)),
                  axis=1, keepdims=True)
    val_ref[...] = m
    idx_ref[...] = idx


def _tc_merge_kernel(logits_ref, u_ref, pval_ref, pidx_ref, out_ref, *,
                     block_rows):
    base_cols = lax.broadcasted_iota(jnp.int32, (block_rows, _CHUNK), 1)

    def body(c, carry):
        acc_val, acc_col = carry
        start = jnp.minimum(c * _CHUNK, _SC_W - _CHUNK)
        cols = base_cols + start
        vals = (_gumbel_from_u(u_ref[:, pl.ds(start, _CHUNK)])
                + logits_ref[:, pl.ds(start, _CHUNK)])
        better = vals > acc_val
        acc_val = jnp.where(better, vals, acc_val)
        acc_col = jnp.where(better, cols, acc_col)
        return acc_val, acc_col

    init = (jnp.full((block_rows, _CHUNK), -jnp.inf, jnp.float32),
            jnp.zeros((block_rows, _CHUNK), jnp.int32))
    acc_val, acc_col = lax.fori_loop(0, pl.cdiv(_SC_W, _CHUNK), body, init,
                                     unroll=8)

    m2 = jnp.max(acc_val, axis=1, keepdims=True)
    idx2 = jnp.min(jnp.where(acc_val == m2, acc_col, jnp.int32(2**31 - 1)),
                   axis=1, keepdims=True)
    win2 = m2 >= pval_ref[...]
    out_ref[...] = jnp.where(win2, idx2, pidx_ref[...])


@jax.jit
def kernel(logits):
    b, vocab = logits.shape
    block_rows = 16
    grid = (b // block_rows,)
    vpad = pl.cdiv(vocab, 128) * 128

    u_sc = _make_sc_uniform()()

    pval, pidx = pl.pallas_call(
        functools.partial(_tc_main_kernel, block_rows=block_rows, vpad=vpad),
        grid=grid,
        in_specs=[pl.BlockSpec((block_rows, vocab), lambda i: (i, 0))],
        out_specs=[pl.BlockSpec((block_rows, 1), lambda i: (i, 0)),
                   pl.BlockSpec((block_rows, 1), lambda i: (i, 0))],
        out_shape=[jax.ShapeDtypeStruct((b, 1), jnp.float32),
                   jax.ShapeDtypeStruct((b, 1), jnp.int32)],
        compiler_params=pltpu.CompilerParams(
            dimension_semantics=("arbitrary",),
        ),
    )(logits)

    out = pl.pallas_call(
        functools.partial(_tc_merge_kernel, block_rows=block_rows),
        grid=grid,
        in_specs=[pl.BlockSpec((block_rows, _SC_W), lambda i: (i, 0)),
                  pl.BlockSpec((block_rows, _SC_W), lambda i: (i, 0)),
                  pl.BlockSpec((block_rows, 1), lambda i: (i, 0)),
                  pl.BlockSpec((block_rows, 1), lambda i: (i, 0))],
        out_specs=pl.BlockSpec((block_rows, 1), lambda i: (i, 0)),
        out_shape=jax.ShapeDtypeStruct((b, 1), jnp.int32),
        compiler_params=pltpu.CompilerParams(
            dimension_semantics=("arbitrary",),
        ),
    )(logits, u_sc, pval, pidx)

    return out[:, 0].astype(jnp.int64)
